# baseline (speedup 1.0000x reference)
import functools

import jax
import jax.numpy as jnp
import numpy as np
from jax.experimental import pallas as pl
from jax.experimental.pallas import tpu as pltpu
from jax.experimental.pallas import tpu_sc as plsc

B, S = 4, 8192
SP = 4
FPDT_CHUNK = 2048
RANK = 1
NCPG = S // FPDT_CHUNK
LOCAL = S // SP
CH = LOCAL // NCPG
TCH = S // CH

PERM = [(g % NCPG) * SP + g // NCPG for g in range(TCH)]
LOCAL_CHUNKS = [PERM[NCPG * RANK + g] for g in range(NCPG)]

_LB_POS = np.tile(
    np.concatenate([np.arange(c * CH, (c + 1) * CH, dtype=np.int32)
                    for c in LOCAL_CHUNKS]),
    (B, 1),
)


@functools.partial(
    pl.kernel,
    mesh=plsc.ScalarSubcoreMesh(axis_name="c", num_cores=1),
    out_type=[
        jax.ShapeDtypeStruct((B, SP, CH), jnp.int32),
        jax.ShapeDtypeStruct((B, SP, CH), jnp.int32),
        jax.ShapeDtypeStruct((B, NCPG, SP, CH), jnp.float32),
    ],
    scratch_types=[
        pltpu.SemaphoreType.DMA,
        pltpu.SemaphoreType.DMA,
        pltpu.SemaphoreType.DMA,
    ],
)
def _fpdt_gather(tok, lab, loss, o_tok, o_lab, o_loss, st_, sl_, sf_):
    c0 = pltpu.async_copy(tok.at[:, :, RANK, :], o_tok, st_)
    c1 = pltpu.async_copy(lab.at[:, :, RANK, :], o_lab, sl_)
    for b in range(B):
        for q in range(NCPG):
            pltpu.async_copy(loss.at[b, :, q, :], o_loss.at[b, q], sf_)
    c0.wait()
    c1.wait()
    pltpu.make_async_copy(o_loss, o_loss, sf_).wait()


def kernel(tokens, labels, loss_mask, attention_mask, position_ids,
           sp_size, sp_rank, fpdt_chunk_size):
    del position_ids, sp_size, sp_rank, fpdt_chunk_size
    o_tok, o_lab, o_loss = _fpdt_gather(
        tokens.reshape(B, SP, NCPG, CH),
        labels.reshape(B, SP, NCPG, CH),
        loss_mask.reshape(B, SP, NCPG, CH),
    )
    return (
        o_tok.reshape(B, LOCAL),
        o_lab.reshape(B, LOCAL),
        o_loss.reshape(B, S),
        attention_mask,
        jnp.asarray(_LB_POS),
    )


# speedup vs baseline: 1.2360x; 1.2360x over previous
import functools

import jax
import jax.numpy as jnp
import numpy as np
from jax import lax
from jax.experimental import pallas as pl
from jax.experimental.pallas import tpu as pltpu
from jax.experimental.pallas import tpu_sc as plsc

B, S = 4, 8192
SP = 4
FPDT_CHUNK = 2048
RANK = 1
NCPG = S // FPDT_CHUNK
LOCAL = S // SP
CH = LOCAL // NCPG
TCH = S // CH

PERM = [(g % NCPG) * SP + g // NCPG for g in range(TCH)]
LOCAL_CHUNKS = [PERM[NCPG * RANK + g] for g in range(NCPG)]

_LB_POS = np.tile(
    np.concatenate([np.arange(c * CH, (c + 1) * CH, dtype=np.int32)
                    for c in LOCAL_CHUNKS]),
    (B, 1),
)

ROWS = 2 * SP + NCPG * SP


@functools.partial(
    pl.kernel,
    mesh=plsc.ScalarSubcoreMesh(axis_name="c", num_cores=1),
    out_type=[
        jax.ShapeDtypeStruct((B, ROWS, CH), jnp.int32),
    ],
    scratch_types=[
        pltpu.VMEM_SHARED((B, SP, CH), jnp.int32),
        pltpu.VMEM_SHARED((B, SP, CH), jnp.int32),
        pltpu.VMEM_SHARED((B, NCPG, SP, CH), jnp.int32),
        pltpu.SemaphoreType.DMA,
        pltpu.SemaphoreType.DMA,
        pltpu.SemaphoreType.DMA,
        pltpu.SemaphoreType.DMA,
    ],
)
def _fpdt_gather(tok, lab, loss, out, tbuf, lbuf, fbuf, st_, sl_, sf_, sg_):
    pltpu.async_copy(tok.at[:, :, RANK, :], tbuf, st_)
    pltpu.async_copy(lab.at[:, :, RANK, :], lbuf, sl_)
    for r in range(SP):
        pltpu.async_copy(loss.at[:, r, :, :], fbuf.at[:, :, r, :], sf_)
    stores = []
    pltpu.make_async_copy(tbuf, tbuf, st_).wait()
    for b in range(B):
        stores.append(pltpu.async_copy(tbuf.at[b], out.at[b, pl.ds(0, SP), :], sg_))
    pltpu.make_async_copy(lbuf, lbuf, sl_).wait()
    for b in range(B):
        stores.append(pltpu.async_copy(lbuf.at[b], out.at[b, pl.ds(SP, SP), :], sg_))
    pltpu.make_async_copy(fbuf, fbuf, sf_).wait()
    for b in range(B):
        for q in range(NCPG):
            stores.append(pltpu.async_copy(
                fbuf.at[b, q], out.at[b, pl.ds(2 * SP + q * SP, SP), :], sg_))
    for st in stores:
        st.wait()


def kernel(tokens, labels, loss_mask, attention_mask, position_ids,
           sp_size, sp_rank, fpdt_chunk_size):
    del position_ids, sp_size, sp_rank, fpdt_chunk_size
    [out] = _fpdt_gather(
        tokens.reshape(B, SP, NCPG, CH),
        labels.reshape(B, SP, NCPG, CH),
        lax.bitcast_convert_type(loss_mask, jnp.int32).reshape(B, SP, NCPG, CH),
    )
    o_tok = out[:, 0:SP, :].reshape(B, LOCAL)
    o_lab = out[:, SP:2 * SP, :].reshape(B, LOCAL)
    o_loss = lax.bitcast_convert_type(
        out[:, 2 * SP:, :].reshape(B, S), jnp.float32)
    return (
        o_tok,
        o_lab,
        o_loss,
        attention_mask,
        jnp.asarray(_LB_POS),
    )
